# Initial kernel scaffold; baseline (speedup 1.0000x reference)
import jax
import jax.numpy as jnp
from jax.experimental import pallas as pl


def kernel(inputs, targets):
    raise NotImplementedError("write your pallas kernel here")


# speedup vs baseline: 2.2108x; 2.2108x over previous
import functools

import jax
import jax.numpy as jnp
from jax.experimental import pallas as pl
from jax.experimental.pallas import tpu as pltpu

_B = 1024
_C = 100000
_M = 0.3
_S = 15.0

_BLOCK_C = 2048
_NBLOCKS = (_C + _BLOCK_C - 1) // _BLOCK_C


def _loss_kernel(x_ref, t_ref, o_ref, m_sc, s_sc, tv_sc):
    i = pl.program_id(0)

    @pl.when(i == 0)
    def _init():
        m_sc[...] = jnp.full((_B, 1), -jnp.inf, jnp.float32)
        s_sc[...] = jnp.zeros((_B, 1), jnp.float32)
        tv_sc[...] = jnp.zeros((_B, 1), jnp.float32)

    x = x_ref[...]
    col = i * _BLOCK_C + jax.lax.broadcasted_iota(jnp.int32, (_B, _BLOCK_C), 1)

    is_last = i == _NBLOCKS - 1
    xm = jnp.where(is_last & (col >= _C), -jnp.inf, x)

    bmax = jnp.max(xm, axis=1, keepdims=True)
    m_old = m_sc[...]
    m_new = jnp.maximum(m_old, bmax)
    s_sc[...] = s_sc[...] * jnp.exp(_S * (m_old - m_new)) + jnp.sum(
        jnp.exp(_S * xm - _S * m_new), axis=1, keepdims=True
    )
    m_sc[...] = m_new

    t = t_ref[...]
    tv_sc[...] += jnp.sum(
        jnp.where(col == t, x, 0.0), axis=1, keepdims=True
    )

    @pl.when(is_last)
    def _finish():
        m = m_sc[...]
        s = s_sc[...]
        tv = tv_sc[...]
        s_adj = s - jnp.exp(_S * (tv - m)) + jnp.exp(_S * (tv - _M - m))
        lse = _S * m + jnp.log(s_adj)
        loss = jnp.mean(lse - _S * (tv - _M))
        o_ref[...] = loss.reshape(1, 1)


@jax.jit
def _amsoftmax_loss(inputs, targets):
    t2d = targets.astype(jnp.int32).reshape(_B, 1)
    out = pl.pallas_call(
        _loss_kernel,
        grid=(_NBLOCKS,),
        in_specs=[
            pl.BlockSpec((_B, _BLOCK_C), lambda i: (0, i)),
            pl.BlockSpec((_B, 1), lambda i: (0, 0)),
        ],
        out_specs=pl.BlockSpec((1, 1), lambda i: (0, 0)),
        out_shape=jax.ShapeDtypeStruct((1, 1), jnp.float32),
        scratch_shapes=[
            pltpu.VMEM((_B, 1), jnp.float32),
            pltpu.VMEM((_B, 1), jnp.float32),
            pltpu.VMEM((_B, 1), jnp.float32),
        ],
        compiler_params=pltpu.CompilerParams(
            dimension_semantics=("arbitrary",),
        ),
    )(inputs, t2d)
    return out[0, 0]


def kernel(inputs, targets):
    return _amsoftmax_loss(inputs, targets)


# speedup vs baseline: 6.6363x; 3.0018x over previous
import functools

import jax
import jax.numpy as jnp
from jax.experimental import pallas as pl
from jax.experimental.pallas import tpu as pltpu

_B = 1024
_C = 100000
_M = 0.3
_S = 15.0

_BLOCK_C = 2048
_NBLOCKS = (_C + _BLOCK_C - 1) // _BLOCK_C


def _loss_kernel(x_ref, t_ref, o_ref, m_sc, s_sc, tv_sc):
    i = pl.program_id(0)

    @pl.when(i == 0)
    def _init():
        m_sc[...] = jnp.full((1, _B), -jnp.inf, jnp.float32)
        s_sc[...] = jnp.zeros((1, _B), jnp.float32)
        tv_sc[...] = jnp.zeros((1, _B), jnp.float32)

    x = x_ref[...]
    row = i * _BLOCK_C + jax.lax.broadcasted_iota(jnp.int32, (_BLOCK_C, _B), 0)

    is_last = i == _NBLOCKS - 1
    xm = jnp.where(is_last & (row >= _C), -jnp.inf, x)

    bmax = jnp.max(xm, axis=0, keepdims=True)
    m_old = m_sc[...]
    m_new = jnp.maximum(m_old, bmax)
    s_sc[...] = s_sc[...] * jnp.exp(_S * (m_old - m_new)) + jnp.sum(
        jnp.exp(_S * xm - _S * m_new), axis=0, keepdims=True
    )
    m_sc[...] = m_new

    t = t_ref[...]
    tv_sc[...] += jnp.sum(jnp.where(row == t, x, 0.0), axis=0, keepdims=True)

    @pl.when(is_last)
    def _finish():
        m = m_sc[...]
        s = s_sc[...]
        tv = tv_sc[...]
        s_adj = s - jnp.exp(_S * (tv - m)) + jnp.exp(_S * (tv - _M - m))
        lse = _S * m + jnp.log(s_adj)
        loss = jnp.mean(lse - _S * (tv - _M))
        o_ref[...] = loss.reshape(1, 1)


@jax.jit
def _amsoftmax_loss(inputs, targets):
    xt = inputs.T
    t2d = targets.astype(jnp.int32).reshape(1, _B)
    out = pl.pallas_call(
        _loss_kernel,
        grid=(_NBLOCKS,),
        in_specs=[
            pl.BlockSpec((_BLOCK_C, _B), lambda i: (i, 0)),
            pl.BlockSpec((1, _B), lambda i: (0, 0)),
        ],
        out_specs=pl.BlockSpec((1, 1), lambda i: (0, 0)),
        out_shape=jax.ShapeDtypeStruct((1, 1), jnp.float32),
        scratch_shapes=[
            pltpu.VMEM((1, _B), jnp.float32),
            pltpu.VMEM((1, _B), jnp.float32),
            pltpu.VMEM((1, _B), jnp.float32),
        ],
        compiler_params=pltpu.CompilerParams(
            dimension_semantics=("arbitrary",),
        ),
    )(xt, t2d)
    return out[0, 0]


def kernel(inputs, targets):
    return _amsoftmax_loss(inputs, targets)


# speedup vs baseline: 6.8133x; 1.0267x over previous
import functools

import jax
import jax.numpy as jnp
from jax.experimental import pallas as pl
from jax.experimental.pallas import tpu as pltpu

_B = 1024
_C = 100000
_M = 0.3
_S = 15.0

_BLOCK_C = 2000
_NBLOCKS = _C // _BLOCK_C


def _loss_kernel(x_ref, t_ref, o_ref, m_sc, s_sc, tv_sc):
    i = pl.program_id(0)

    @pl.when(i == 0)
    def _init():
        m_sc[...] = jnp.full((1, _B), -jnp.inf, jnp.float32)
        s_sc[...] = jnp.zeros((1, _B), jnp.float32)
        tv_sc[...] = jnp.zeros((1, _B), jnp.float32)

    x = x_ref[...]
    row = i * _BLOCK_C + jax.lax.broadcasted_iota(jnp.int32, (_BLOCK_C, _B), 0)

    bmax = jnp.max(x, axis=0, keepdims=True)
    m_old = m_sc[...]
    m_new = jnp.maximum(m_old, bmax)
    s_sc[...] = s_sc[...] * jnp.exp(_S * (m_old - m_new)) + jnp.sum(
        jnp.exp(_S * x - _S * m_new), axis=0, keepdims=True
    )
    m_sc[...] = m_new

    t = t_ref[...]
    tv_sc[...] += jnp.sum(jnp.where(row == t, x, 0.0), axis=0, keepdims=True)

    @pl.when(i == _NBLOCKS - 1)
    def _finish():
        m = m_sc[...]
        s = s_sc[...]
        tv = tv_sc[...]
        s_adj = s - jnp.exp(_S * (tv - m)) + jnp.exp(_S * (tv - _M - m))
        lse = _S * m + jnp.log(s_adj)
        loss = jnp.mean(lse - _S * (tv - _M))
        o_ref[...] = loss.reshape(1, 1)


@jax.jit
def _amsoftmax_loss(inputs, targets):
    xt = inputs.T
    t2d = targets.astype(jnp.int32).reshape(1, _B)
    out = pl.pallas_call(
        _loss_kernel,
        grid=(_NBLOCKS,),
        in_specs=[
            pl.BlockSpec((_BLOCK_C, _B), lambda i: (i, 0)),
            pl.BlockSpec((1, _B), lambda i: (0, 0)),
        ],
        out_specs=pl.BlockSpec((1, 1), lambda i: (0, 0)),
        out_shape=jax.ShapeDtypeStruct((1, 1), jnp.float32),
        scratch_shapes=[
            pltpu.VMEM((1, _B), jnp.float32),
            pltpu.VMEM((1, _B), jnp.float32),
            pltpu.VMEM((1, _B), jnp.float32),
        ],
        compiler_params=pltpu.CompilerParams(
            dimension_semantics=("arbitrary",),
        ),
    )(xt, t2d)
    return out[0, 0]


def kernel(inputs, targets):
    return _amsoftmax_loss(inputs, targets)
